# baseline (speedup 1.0000x reference)
import functools

import jax
import jax.numpy as jnp
from jax import lax
from jax.experimental import pallas as pl
from jax.experimental.pallas import tpu as pltpu
from jax.experimental.pallas import tpu_sc as plsc

_K = 8192
_D = 32
_Z = 512
_KC = 1024
_STEPS = _K // _KC


def _score_body(z_ref, ct_ref, tok_ref, rep_ref, zt_ref, mval_ref, midx_ref):
    i = pl.program_id(0)

    @pl.when(i == 0)
    def _():
        mval_ref[...] = jnp.full((1, _Z), jnp.inf, jnp.float32)
        midx_ref[...] = jnp.zeros((1, _Z), jnp.float32)
        zt_ref[...] = -2.0 * z_ref[...].T

    ct = ct_ref[...]
    rep_ref[...] = jnp.concatenate([ct.T] * 4, axis=1)
    cn = jnp.sum(ct * ct, axis=0)[:, None]
    zc = lax.dot_general(
        ct, zt_ref[...], (((0,), (0,)), ((), ())),
        preferred_element_type=jnp.float32,
        precision=lax.Precision.HIGHEST,
    )
    s = zc + cn
    lmin = jnp.min(s, axis=0, keepdims=True)
    iota = lax.broadcasted_iota(jnp.int32, (_KC, _Z), 0).astype(jnp.float32)
    li = jnp.min(jnp.where(s == lmin, iota, 3.0e38), axis=0, keepdims=True)
    li = li + jnp.float32(i * _KC)
    better = lmin < mval_ref[...]
    mval_ref[...] = jnp.where(better, lmin, mval_ref[...])
    midx_ref[...] = jnp.where(better, li, midx_ref[...])

    @pl.when(i == _STEPS - 1)
    def _():
        tok_ref[...] = midx_ref[...].astype(jnp.int32)


def _tokens_and_table(z, ct):
    return pl.pallas_call(
        _score_body,
        grid=(_STEPS,),
        in_specs=[
            pl.BlockSpec((_Z, _D), lambda i: (0, 0)),
            pl.BlockSpec((_D, _KC), lambda i: (0, i)),
        ],
        out_specs=[
            pl.BlockSpec((1, _Z), lambda i: (0, 0)),
            pl.BlockSpec((_KC, 128), lambda i: (i, 0)),
        ],
        out_shape=[
            jax.ShapeDtypeStruct((1, _Z), jnp.int32),
            jax.ShapeDtypeStruct((_K, 128), jnp.float32),
        ],
        scratch_shapes=[
            pltpu.VMEM((_D, _Z), jnp.float32),
            pltpu.VMEM((1, _Z), jnp.float32),
            pltpu.VMEM((1, _Z), jnp.float32),
        ],
    )(z, ct)


def _sc_gather(table, tokens):
    info = plsc.get_sparse_core_info()
    nc, ns = 1, info.num_subcores
    nw = nc * ns
    b_per_w = _Z // nw
    mesh = plsc.VectorSubcoreMesh(core_axis_name="c", subcore_axis_name="s", num_cores=1)

    @functools.partial(
        pl.kernel, mesh=mesh,
        out_type=jax.ShapeDtypeStruct((_Z, _D), jnp.float32),
        scratch_types=[
            pltpu.VMEM((b_per_w,), jnp.int32),
            pltpu.VMEM((b_per_w, 128), jnp.float32),
            pltpu.VMEM((b_per_w, _D), jnp.float32),
            pltpu.SemaphoreType.DMA,
        ],
    )
    def k(table_hbm, idx_hbm, out_hbm, idx_v, lines_v, out_v, sem):
        wid = lax.axis_index("s") * nc + lax.axis_index("c")
        base = wid * b_per_w
        pltpu.sync_copy(idx_hbm.at[pl.ds(base, b_per_w)], idx_v)
        pltpu.async_copy(table_hbm.at[idx_v], lines_v, sem).wait()
        for r in range(b_per_w):
            for j in range(_D // 16):
                out_v[r, pl.ds(16 * j, 16)] = lines_v[r, pl.ds(16 * j, 16)]
        pltpu.sync_copy(out_v, out_hbm.at[pl.ds(base, b_per_w)])

    return k(table, tokens)


def kernel(inputs, codebook, training):
    del training
    b, h, w, d = inputs.shape
    z = inputs.reshape(b * h * w, d)
    tokens, table = _tokens_and_table(z, codebook.T)
    emb = _sc_gather(table, tokens.reshape(_Z))
    return emb.reshape(b, h, w, d)


# speedup vs baseline: 1.0609x; 1.0609x over previous
import functools

import jax
import jax.numpy as jnp
from jax import lax
from jax.experimental import pallas as pl
from jax.experimental.pallas import tpu as pltpu
from jax.experimental.pallas import tpu_sc as plsc

_K = 8192
_D = 32
_Z = 512
_KC = 2048
_STEPS = _K // _KC


def _score_body(z_ref, ct_ref, tok_ref, rep_ref, zt_ref, mval_ref, midx_ref):
    i = pl.program_id(0)

    @pl.when(i == 0)
    def _():
        mval_ref[...] = jnp.full((1, _Z), jnp.inf, jnp.float32)
        midx_ref[...] = jnp.zeros((1, _Z), jnp.float32)
        zt_ref[...] = -2.0 * z_ref[...].T

    ct = ct_ref[...]
    rep_ref[...] = jnp.concatenate([ct.T] * 4, axis=1)
    cn = jnp.sum(ct * ct, axis=0)[:, None]
    zc = lax.dot_general(
        ct, zt_ref[...], (((0,), (0,)), ((), ())),
        preferred_element_type=jnp.float32,
        precision=lax.Precision.HIGHEST,
    )
    s = zc + cn
    lmin = jnp.min(s, axis=0, keepdims=True)
    li = jnp.argmin(s, axis=0)[None, :].astype(jnp.float32)
    li = li + jnp.float32(i * _KC)
    better = lmin < mval_ref[...]
    mval_ref[...] = jnp.where(better, lmin, mval_ref[...])
    midx_ref[...] = jnp.where(better, li, midx_ref[...])

    @pl.when(i == _STEPS - 1)
    def _():
        tok_ref[...] = midx_ref[...].astype(jnp.int32)


def _tokens_and_table(z, ct):
    return pl.pallas_call(
        _score_body,
        grid=(_STEPS,),
        in_specs=[
            pl.BlockSpec((_Z, _D), lambda i: (0, 0)),
            pl.BlockSpec((_D, _KC), lambda i: (0, i)),
        ],
        out_specs=[
            pl.BlockSpec((1, _Z), lambda i: (0, 0)),
            pl.BlockSpec((_KC, 128), lambda i: (i, 0)),
        ],
        out_shape=[
            jax.ShapeDtypeStruct((1, _Z), jnp.int32),
            jax.ShapeDtypeStruct((_K, 128), jnp.float32),
        ],
        scratch_shapes=[
            pltpu.VMEM((_D, _Z), jnp.float32),
            pltpu.VMEM((1, _Z), jnp.float32),
            pltpu.VMEM((1, _Z), jnp.float32),
        ],
    )(z, ct)


def _sc_gather(table, tokens):
    info = plsc.get_sparse_core_info()
    nc, ns = 1, info.num_subcores
    nw = nc * ns
    b_per_w = _Z // nw
    mesh = plsc.VectorSubcoreMesh(core_axis_name="c", subcore_axis_name="s", num_cores=1)

    @functools.partial(
        pl.kernel, mesh=mesh,
        out_type=jax.ShapeDtypeStruct((_Z, _D), jnp.float32),
        scratch_types=[
            pltpu.VMEM((b_per_w,), jnp.int32),
            pltpu.VMEM((b_per_w, 128), jnp.float32),
            pltpu.VMEM((b_per_w, _D), jnp.float32),
            pltpu.SemaphoreType.DMA,
        ],
    )
    def k(table_hbm, idx_hbm, out_hbm, idx_v, lines_v, out_v, sem):
        wid = lax.axis_index("s") * nc + lax.axis_index("c")
        base = wid * b_per_w
        pltpu.sync_copy(idx_hbm.at[pl.ds(base, b_per_w)], idx_v)
        pltpu.async_copy(table_hbm.at[idx_v], lines_v, sem).wait()
        for r in range(b_per_w):
            for j in range(_D // 16):
                out_v[r, pl.ds(16 * j, 16)] = lines_v[r, pl.ds(16 * j, 16)]
        pltpu.sync_copy(out_v, out_hbm.at[pl.ds(base, b_per_w)])

    return k(table, tokens)


def kernel(inputs, codebook, training):
    del training
    b, h, w, d = inputs.shape
    z = inputs.reshape(b * h * w, d)
    tokens, table = _tokens_and_table(z, codebook.T)
    emb = _sc_gather(table, tokens.reshape(_Z))
    return emb.reshape(b, h, w, d)


# speedup vs baseline: 1.0747x; 1.0130x over previous
import functools

import jax
import jax.numpy as jnp
from jax import lax
from jax.experimental import pallas as pl
from jax.experimental.pallas import tpu as pltpu
from jax.experimental.pallas import tpu_sc as plsc

_K = 8192
_D = 32
_Z = 512
_KC = 2048
_STEPS = _K // _KC


def _score_body(z_ref, ct_ref, tok_ref, rep_ref, zt_ref, mval_ref, midx_ref):
    i = pl.program_id(0)

    @pl.when(i == 0)
    def _():
        mval_ref[...] = jnp.full((1, _Z), jnp.inf, jnp.float32)
        midx_ref[...] = jnp.zeros((1, _Z), jnp.float32)
        zt_ref[...] = -2.0 * z_ref[...].T

    ct = ct_ref[...]
    rep_ref[:, : _D] = ct.T
    cn = jnp.sum(ct * ct, axis=0)[:, None]
    zc = lax.dot_general(
        ct, zt_ref[...], (((0,), (0,)), ((), ())),
        preferred_element_type=jnp.float32,
        precision=lax.Precision.HIGHEST,
    )
    s = zc + cn
    lmin = jnp.min(s, axis=0, keepdims=True)
    li = jnp.argmin(s, axis=0)[None, :].astype(jnp.float32)
    li = li + jnp.float32(i * _KC)
    better = lmin < mval_ref[...]
    mval_ref[...] = jnp.where(better, lmin, mval_ref[...])
    midx_ref[...] = jnp.where(better, li, midx_ref[...])

    @pl.when(i == _STEPS - 1)
    def _():
        tok_ref[...] = midx_ref[...].astype(jnp.int32)


def _tokens_and_table(z, ct):
    return pl.pallas_call(
        _score_body,
        grid=(_STEPS,),
        in_specs=[
            pl.BlockSpec((_Z, _D), lambda i: (0, 0)),
            pl.BlockSpec((_D, _KC), lambda i: (0, i)),
        ],
        out_specs=[
            pl.BlockSpec((1, _Z), lambda i: (0, 0)),
            pl.BlockSpec((_KC, 128), lambda i: (i, 0)),
        ],
        out_shape=[
            jax.ShapeDtypeStruct((1, _Z), jnp.int32),
            jax.ShapeDtypeStruct((_K, 128), jnp.float32),
        ],
        scratch_shapes=[
            pltpu.VMEM((_D, _Z), jnp.float32),
            pltpu.VMEM((1, _Z), jnp.float32),
            pltpu.VMEM((1, _Z), jnp.float32),
        ],
    )(z, ct)


def _sc_gather(table, tokens):
    info = plsc.get_sparse_core_info()
    nc, ns = 1, info.num_subcores
    nw = nc * ns
    b_per_w = _Z // nw
    mesh = plsc.VectorSubcoreMesh(core_axis_name="c", subcore_axis_name="s", num_cores=1)

    @functools.partial(
        pl.kernel, mesh=mesh,
        out_type=jax.ShapeDtypeStruct((_Z, _D), jnp.float32),
        scratch_types=[
            pltpu.VMEM((b_per_w,), jnp.int32),
            pltpu.VMEM((b_per_w, 128), jnp.float32),
            pltpu.VMEM((b_per_w, _D), jnp.float32),
            pltpu.SemaphoreType.DMA,
        ],
    )
    def k(table_hbm, idx_hbm, out_hbm, idx_v, lines_v, out_v, sem):
        wid = lax.axis_index("s") * nc + lax.axis_index("c")
        base = wid * b_per_w
        pltpu.sync_copy(idx_hbm.at[pl.ds(base, b_per_w)], idx_v)
        pltpu.async_copy(table_hbm.at[idx_v], lines_v, sem).wait()
        for r in range(b_per_w):
            for j in range(_D // 16):
                out_v[r, pl.ds(16 * j, 16)] = lines_v[r, pl.ds(16 * j, 16)]
        pltpu.sync_copy(out_v, out_hbm.at[pl.ds(base, b_per_w)])

    return k(table, tokens)


def kernel(inputs, codebook, training):
    del training
    b, h, w, d = inputs.shape
    z = inputs.reshape(b * h * w, d)
    tokens, table = _tokens_and_table(z, codebook.T)
    emb = _sc_gather(table, tokens.reshape(_Z))
    return emb.reshape(b, h, w, d)
